# baseline (speedup 1.0000x reference)
import functools

import jax
import jax.numpy as jnp
from jax import lax
from jax.experimental import pallas as pl
from jax.experimental.pallas import tpu as pltpu
from jax.experimental.pallas import tpu_sc as plsc

N = 10000
E = 160000
D = 256
H = 128
OUT = 128
EPS = 1e-5
NC = 2
NS = 16
K = 128
KD = 128
E_TOT = E + N
EPB = NC * NS * KD
E_PAD = ((E_TOT + EPB - 1) // EPB) * EPB
AGG_PT = E_PAD // NS
AGG_NB = AGG_PT // K
DEG_PW = E_PAD // (NC * NS)
DEG_NB = DEG_PW // KD
NROWS = 10112
RPT = NROWS // NS
BN = 2000
GRID = N // BN
SBN = 256
SGRID = (NROWS + SBN - 1) // SBN
NBUF = 2

def _zero_rows(buf, nrows, ncols):
    def body_r(r, carry):
        def body_c(cc, carry2):
            buf[r, pl.ds(cc * 16, 16)] = jnp.zeros((16,), jnp.float32)
            return carry2
        return lax.fori_loop(0, ncols // 16, body_c, carry)
    lax.fori_loop(0, nrows, body_r, 0)


def _zero_stripe(zsrc, acc, base, total):
    zb = zsrc.shape[0]
    k0 = 0
    while k0 < total:
        sz = min(zb, total - k0)
        pltpu.sync_copy(zsrc.at[pl.ds(0, sz)], acc.at[pl.ds(base + k0, sz)])
        k0 += sz


def _deg_body(dst_hbm, out_hbm, idx_d, ones, zbuf, acc):
    c = lax.axis_index("c")
    s = lax.axis_index("s")

    def fill_r(r, carry):
        ones[r] = jnp.full((16,), 1.0, jnp.float32)
        zbuf[r] = jnp.zeros((16,), jnp.float32)
        return carry
    lax.fori_loop(0, KD, fill_r, 0)

    base_r = s * RPT
    _zero_stripe(zbuf, acc, base_r, RPT)
    pltpu.sync_copy(dst_hbm.at[s], idx_d)
    plsc.subcore_barrier()

    def body(j, carry):
        pltpu.sync_copy(ones, acc.at[idx_d.at[j]], add=True)
        return carry
    lax.fori_loop(c * DEG_NB, (c + 1) * DEG_NB, body, 0)

    plsc.subcore_barrier()
    pltpu.sync_copy(acc.at[pl.ds(base_r, RPT)],
                    out_hbm.at[pl.ds(c * NROWS + base_r, RPT)])


def _agg_body(hs_hbm, srcs_hbm, dst_hbm, out_hbm, idx_s, dring, rows, acc,
              gsem, dsem):
    c = lax.axis_index("c")
    s = lax.axis_index("s")

    pltpu.sync_copy(srcs_hbm.at[c * NS + s], idx_s)

    rows0 = rows.at[0]
    _zero_rows(rows0, K, H)
    base_r = s * RPT
    _zero_stripe(rows0, acc, base_r, RPT)
    plsc.subcore_barrier()

    for b in range(NBUF):
        pltpu.async_copy(hs_hbm.at[idx_s.at[b]], rows.at[b], gsem)
        pltpu.async_copy(dst_hbm.at[s, b], dring.at[b], dsem)

    def outer(o, carry):
        for b in range(NBUF):
            j = o * NBUF + b
            pltpu.make_async_copy(hs_hbm.at[idx_s.at[j]], rows.at[b],
                                  gsem).wait()
            pltpu.make_async_copy(dst_hbm.at[s, j], dring.at[b], dsem).wait()
            pltpu.sync_copy(rows.at[b], acc.at[dring.at[b]], add=True)
            nj = j + NBUF

            @pl.when(nj < AGG_NB)
            def _():
                pltpu.async_copy(hs_hbm.at[idx_s.at[nj]], rows.at[b], gsem)
                pltpu.async_copy(dst_hbm.at[s, nj], dring.at[b], dsem)
        return carry
    lax.fori_loop(0, AGG_NB // NBUF, outer, 0)

    plsc.subcore_barrier()
    pltpu.sync_copy(acc.at[pl.ds(base_r, RPT)],
                    out_hbm.at[pl.ds(c * NROWS + base_r, RPT)])


@functools.cache
def _sc_calls():
    mesh = plsc.VectorSubcoreMesh(
        core_axis_name="c", subcore_axis_name="s",
        num_cores=NC, num_subcores=NS)
    deg = pl.kernel(
        _deg_body,
        out_type=jax.ShapeDtypeStruct((NC * NROWS, 16), jnp.float32),
        mesh=mesh,
        scratch_types=[
            pltpu.VMEM((AGG_NB, KD), jnp.int32),
            pltpu.VMEM((KD, 16), jnp.float32),
            pltpu.VMEM((KD, 16), jnp.float32),
            pltpu.VMEM_SHARED((NROWS, 16), jnp.float32),
        ],
    )
    agg = pl.kernel(
        _agg_body,
        out_type=jax.ShapeDtypeStruct((NC * NROWS, H), jnp.float32),
        mesh=mesh,
        scratch_types=[
            pltpu.VMEM((AGG_NB, K), jnp.int32),
            pltpu.VMEM((NBUF, K), jnp.int32),
            pltpu.VMEM((NBUF, K, H), jnp.float32),
            pltpu.VMEM_SHARED((NROWS, H), jnp.float32),
            pltpu.SemaphoreType.DMA,
            pltpu.SemaphoreType.DMA,
        ],
    )
    return deg, agg


def _prep_body(x_ref, w0_ref, pw_ref, deg_ref, hs_ref, hproj_ref, dinv_ref):
    deg = deg_ref[0] + deg_ref[1]
    dinv = lax.rsqrt(deg[:, 0:1])
    x = x_ref[...]
    hw = jnp.dot(x, w0_ref[...], preferred_element_type=jnp.float32)
    hp = jnp.dot(x, pw_ref[...], preferred_element_type=jnp.float32)
    hs = hw * dinv
    hs_ref[0] = hs[:, :H]
    hs_ref[1] = hs[:, H:]
    hproj_ref[0] = hp[:, :H]
    hproj_ref[1] = hp[:, H:]
    dinv_ref[...] = dinv


_prep_call = pl.pallas_call(
    _prep_body,
    grid=(GRID,),
    in_specs=[
        pl.BlockSpec((BN, D), lambda i: (i, 0)),
        pl.BlockSpec((D, D), lambda i: (0, 0)),
        pl.BlockSpec((D, D), lambda i: (0, 0)),
        pl.BlockSpec((2, BN, 16), lambda i: (0, i, 0)),
    ],
    out_specs=[
        pl.BlockSpec((2, BN, H), lambda i: (0, i, 0)),
        pl.BlockSpec((2, BN, H), lambda i: (0, i, 0)),
        pl.BlockSpec((BN, 1), lambda i: (i, 0)),
    ],
    out_shape=[
        jax.ShapeDtypeStruct((2, N, H), jnp.float32),
        jax.ShapeDtypeStruct((2, N, H), jnp.float32),
        jax.ShapeDtypeStruct((N, 1), jnp.float32),
    ],
)


def _halves(agg_ref, dinv_ref, cb_ref):
    dinv = dinv_ref[...]
    return [agg_ref[h] * dinv + cb_ref[h] for h in (0, 1)]


def _accum_stats(p, i, ch, ssum, ssq):
    @pl.when(p == 0)
    def _():
        s1 = [jnp.sum(c, axis=0, keepdims=True) for c in ch]
        s2 = [jnp.sum(c * c, axis=0, keepdims=True) for c in ch]

        @pl.when(i == 0)
        def _():
            for h in (0, 1):
                ssum[h] = s1[h]
                ssq[h] = s2[h]

        @pl.when(i > 0)
        def _():
            for h in (0, 1):
                ssum[h] += s1[h]
                ssq[h] += s2[h]


def _normed_h(ch, ssum, ssq, nw_ref, nb_ref, nm_ref, res_ref):
    out = []
    for h in (0, 1):
        mean = ssum[h] * (1.0 / N)
        msq = ssq[h] * (1.0 / N)
        ctr = mean * nm_ref[h]
        var = msq - 2.0 * ctr * mean + ctr * ctr
        a = nw_ref[h] * lax.rsqrt(var + EPS)
        y = (ch[h] - ctr) * a + nb_ref[h]
        out.append(jnp.maximum(y, 0.0) + res_ref[h])
    return out


def _layer_body(agg_ref, dinv_ref, cb_ref, nw_ref, nb_ref, nm_ref, res_ref,
                wn_ref, h_ref, hs_ref, ssum, ssq):
    p = pl.program_id(0)
    i = pl.program_id(1)
    ch = _halves(agg_ref, dinv_ref, cb_ref)
    _accum_stats(p, i, ch, ssum, ssq)

    @pl.when(p == 1)
    def _():
        hn = _normed_h(ch, ssum, ssq, nw_ref, nb_ref, nm_ref, res_ref)
        h_ref[0] = hn[0]
        h_ref[1] = hn[1]
        w = wn_ref[...]
        hw = (jnp.dot(hn[0], w[:H], preferred_element_type=jnp.float32)
              + jnp.dot(hn[1], w[H:], preferred_element_type=jnp.float32))
        hs = hw * dinv_ref[...]
        hs_ref[0] = hs[:, :H]
        hs_ref[1] = hs[:, H:]


_VEC = pl.BlockSpec((2, 1, H), lambda p, i: (0, 0, 0))

_layer_call = pl.pallas_call(
    _layer_body,
    grid=(2, GRID),
    in_specs=[
        pl.BlockSpec((2, BN, H), lambda p, i: (0, i, 0)),
        pl.BlockSpec((BN, 1), lambda p, i: (i, 0)),
        _VEC, _VEC, _VEC, _VEC,
        pl.BlockSpec((2, BN, H), lambda p, i: (0, p * i, 0)),
        pl.BlockSpec((D, D), lambda p, i: (0, 0)),
    ],
    out_specs=[
        pl.BlockSpec((2, BN, H), lambda p, i: (0, p * i, 0)),
        pl.BlockSpec((2, BN, H), lambda p, i: (0, p * i, 0)),
    ],
    out_shape=[
        jax.ShapeDtypeStruct((2, N, H), jnp.float32),
        jax.ShapeDtypeStruct((2, N, H), jnp.float32),
    ],
    scratch_shapes=[
        pltpu.VMEM((2, 1, H), jnp.float32),
        pltpu.VMEM((2, 1, H), jnp.float32),
    ],
)


def _final_body(agg_ref, dinv_ref, cb_ref, nw_ref, nb_ref, nm_ref, res_ref,
                l1_ref, l1b_ref, l2_ref, l2b_ref, out_ref, ssum, ssq):
    p = pl.program_id(0)
    i = pl.program_id(1)
    ch = _halves(agg_ref, dinv_ref, cb_ref)
    _accum_stats(p, i, ch, ssum, ssq)

    @pl.when(p == 1)
    def _():
        hn = _normed_h(ch, ssum, ssq, nw_ref, nb_ref, nm_ref, res_ref)
        l1 = l1_ref[...]
        t = jnp.maximum(
            jnp.dot(hn[0], l1[:H], preferred_element_type=jnp.float32)
            + jnp.dot(hn[1], l1[H:], preferred_element_type=jnp.float32)
            + l1b_ref[...], 0.0)
        out_ref[...] = (
            jnp.dot(t, l2_ref[...], preferred_element_type=jnp.float32)
            + l2b_ref[...])


_final_call = pl.pallas_call(
    _final_body,
    grid=(2, GRID),
    in_specs=[
        pl.BlockSpec((2, BN, H), lambda p, i: (0, i, 0)),
        pl.BlockSpec((BN, 1), lambda p, i: (i, 0)),
        _VEC, _VEC, _VEC, _VEC,
        pl.BlockSpec((2, BN, H), lambda p, i: (0, p * i, 0)),
        pl.BlockSpec((D, D), lambda p, i: (0, 0)),
        pl.BlockSpec((1, D), lambda p, i: (0, 0)),
        pl.BlockSpec((D, OUT), lambda p, i: (0, 0)),
        pl.BlockSpec((1, OUT), lambda p, i: (0, 0)),
    ],
    out_specs=[pl.BlockSpec((BN, OUT), lambda p, i: (p * i, 0))],
    out_shape=[jax.ShapeDtypeStruct((N, OUT), jnp.float32)],
    scratch_shapes=[
        pltpu.VMEM((2, 1, H), jnp.float32),
        pltpu.VMEM((2, 1, H), jnp.float32),
    ],
)


def kernel(x, edge_index, conv_Ws, conv_bs, norm_ws, norm_bs, norm_ms,
           proj_W, lin_Ws, lin_bs):
    src = edge_index[0].astype(jnp.int32)
    dst = edge_index[1].astype(jnp.int32)
    ar = jnp.arange(N, dtype=jnp.int32)
    pad = E_PAD - E_TOT
    src_p = jnp.concatenate([src, ar, jnp.zeros((pad,), jnp.int32)])
    dst_p = jnp.concatenate([dst, ar, jnp.full((pad,), N, jnp.int32)])
    srcs = jnp.concatenate([src_p, src_p + N])

    _deg_call, _agg_call = _sc_calls()
    srcs2 = srcs.reshape(NC * NS, AGG_NB, K)
    dst2 = dst_p.reshape(NS, AGG_NB, K)
    deg2 = _deg_call(dst2).reshape(NC, NROWS, 16)
    hs, hproj, dinv = _prep_call(x, conv_Ws[0], proj_W, deg2)

    res = hproj
    out = None
    for i in range(3):
        agg = _agg_call(hs.reshape(NC * N, H), srcs2, dst2)
        agg = agg.reshape(NC, NROWS, H)
        cb = conv_bs[i].reshape(2, 1, H)
        nw = norm_ws[i].reshape(2, 1, H)
        nb = norm_bs[i].reshape(2, 1, H)
        nm = norm_ms[i].reshape(2, 1, H)
        if i < 2:
            res, hs = _layer_call(agg, dinv, cb, nw, nb, nm, res,
                                  conv_Ws[i + 1])
        else:
            (out,) = _final_call(agg, dinv, cb, nw, nb, nm, res,
                                 lin_Ws[0], lin_bs[0].reshape(1, D),
                                 lin_Ws[1], lin_bs[1].reshape(1, OUT))
    return out


# speedup vs baseline: 1.0382x; 1.0382x over previous
import functools

import jax
import jax.numpy as jnp
from jax import lax
from jax.experimental import pallas as pl
from jax.experimental.pallas import tpu as pltpu
from jax.experimental.pallas import tpu_sc as plsc

N = 10000
E = 160000
D = 256
H = 128
OUT = 128
EPS = 1e-5
NC = 2
NS = 16
K = 128
KD = 128
E_TOT = E + N
EPB = NC * NS * KD
E_PAD = ((E_TOT + EPB - 1) // EPB) * EPB
AGG_PT = E_PAD // NS
AGG_NB = AGG_PT // K
DEG_PW = E_PAD // (NC * NS)
DEG_NB = DEG_PW // KD
NROWS = 10112
RPT = NROWS // NS
BN = 2000
GRID = N // BN
SBN = 256
SGRID = (NROWS + SBN - 1) // SBN
NBUF = 2

def _zero_rows(buf, nrows, ncols):
    def body_r(r, carry):
        def body_c(cc, carry2):
            buf[r, pl.ds(cc * 16, 16)] = jnp.zeros((16,), jnp.float32)
            return carry2
        return lax.fori_loop(0, ncols // 16, body_c, carry)
    lax.fori_loop(0, nrows, body_r, 0)


def _zero_stripe(zsrc, acc, base, total):
    zb = zsrc.shape[0]
    k0 = 0
    while k0 < total:
        sz = min(zb, total - k0)
        pltpu.sync_copy(zsrc.at[pl.ds(0, sz)], acc.at[pl.ds(base + k0, sz)])
        k0 += sz


def _deg_body(dst_hbm, out_hbm, idx_d, ones, zbuf, acc):
    c = lax.axis_index("c")
    s = lax.axis_index("s")
    w = s * NC + c

    def fill_r(r, carry):
        ones[r] = jnp.full((16,), 1.0, jnp.float32)
        zbuf[r] = jnp.zeros((16,), jnp.float32)
        return carry
    lax.fori_loop(0, KD, fill_r, 0)

    base_r = s * RPT
    _zero_stripe(zbuf, acc, base_r, RPT)
    plsc.subcore_barrier()

    eb = w * DEG_PW

    def body(j, carry):
        pltpu.sync_copy(dst_hbm.at[pl.ds(eb + j * KD, KD)], idx_d)
        pltpu.sync_copy(ones, acc.at[idx_d], add=True)
        return carry
    lax.fori_loop(0, DEG_NB, body, 0)

    plsc.subcore_barrier()
    pltpu.sync_copy(acc.at[pl.ds(base_r, RPT)],
                    out_hbm.at[pl.ds(c * NROWS + base_r, RPT)])


def _agg_body(hs_hbm, srcs_hbm, dst_hbm, out_hbm, idx_s, dring, rows, acc,
              gsem, dsem):
    c = lax.axis_index("c")
    s = lax.axis_index("s")

    pltpu.sync_copy(srcs_hbm.at[c * NS + s], idx_s)

    rows0 = rows.at[0]
    _zero_rows(rows0, K, H)
    base_r = s * RPT
    _zero_stripe(rows0, acc, base_r, RPT)
    plsc.subcore_barrier()

    for b in range(NBUF):
        pltpu.async_copy(hs_hbm.at[idx_s.at[b]], rows.at[b], gsem)
        pltpu.async_copy(dst_hbm.at[s, b], dring.at[b], dsem)

    def outer(o, carry):
        for b in range(NBUF):
            j = o * NBUF + b
            pltpu.make_async_copy(hs_hbm.at[idx_s.at[j]], rows.at[b],
                                  gsem).wait()
            pltpu.make_async_copy(dst_hbm.at[s, j], dring.at[b], dsem).wait()
            pltpu.sync_copy(rows.at[b], acc.at[dring.at[b]], add=True)
            nj = j + NBUF

            @pl.when(nj < AGG_NB)
            def _():
                pltpu.async_copy(hs_hbm.at[idx_s.at[nj]], rows.at[b], gsem)
                pltpu.async_copy(dst_hbm.at[s, nj], dring.at[b], dsem)
        return carry
    lax.fori_loop(0, AGG_NB // NBUF, outer, 0)

    plsc.subcore_barrier()
    pltpu.sync_copy(acc.at[pl.ds(base_r, RPT)],
                    out_hbm.at[pl.ds(c * NROWS + base_r, RPT)])


@functools.cache
def _sc_calls():
    mesh = plsc.VectorSubcoreMesh(
        core_axis_name="c", subcore_axis_name="s",
        num_cores=NC, num_subcores=NS)
    deg = pl.kernel(
        _deg_body,
        out_type=jax.ShapeDtypeStruct((NC * NROWS, 16), jnp.float32),
        mesh=mesh,
        scratch_types=[
            pltpu.VMEM((KD,), jnp.int32),
            pltpu.VMEM((KD, 16), jnp.float32),
            pltpu.VMEM((KD, 16), jnp.float32),
            pltpu.VMEM_SHARED((NROWS, 16), jnp.float32),
        ],
    )
    agg = pl.kernel(
        _agg_body,
        out_type=jax.ShapeDtypeStruct((NC * NROWS, H), jnp.float32),
        mesh=mesh,
        scratch_types=[
            pltpu.VMEM((AGG_NB, K), jnp.int32),
            pltpu.VMEM((NBUF, K), jnp.int32),
            pltpu.VMEM((NBUF, K, H), jnp.float32),
            pltpu.VMEM_SHARED((NROWS, H), jnp.float32),
            pltpu.SemaphoreType.DMA,
            pltpu.SemaphoreType.DMA,
        ],
    )
    return deg, agg


def _prep_body(x_ref, w0_ref, pw_ref, deg_ref, hs_ref, hproj_ref, dinv_ref):
    deg = deg_ref[0] + deg_ref[1]
    dinv = lax.rsqrt(deg[:, 0:1])
    x = x_ref[...]
    hw = jnp.dot(x, w0_ref[...], preferred_element_type=jnp.float32)
    hp = jnp.dot(x, pw_ref[...], preferred_element_type=jnp.float32)
    hs = hw * dinv
    hs_ref[0] = hs[:, :H]
    hs_ref[1] = hs[:, H:]
    hproj_ref[0] = hp[:, :H]
    hproj_ref[1] = hp[:, H:]
    dinv_ref[...] = dinv


_prep_call = pl.pallas_call(
    _prep_body,
    grid=(GRID,),
    in_specs=[
        pl.BlockSpec((BN, D), lambda i: (i, 0)),
        pl.BlockSpec((D, D), lambda i: (0, 0)),
        pl.BlockSpec((D, D), lambda i: (0, 0)),
        pl.BlockSpec((2, BN, 16), lambda i: (0, i, 0)),
    ],
    out_specs=[
        pl.BlockSpec((2, BN, H), lambda i: (0, i, 0)),
        pl.BlockSpec((2, BN, H), lambda i: (0, i, 0)),
        pl.BlockSpec((BN, 1), lambda i: (i, 0)),
    ],
    out_shape=[
        jax.ShapeDtypeStruct((2, N, H), jnp.float32),
        jax.ShapeDtypeStruct((2, N, H), jnp.float32),
        jax.ShapeDtypeStruct((N, 1), jnp.float32),
    ],
)


def _halves(agg_ref, dinv_ref, cb_ref):
    dinv = dinv_ref[...]
    return [agg_ref[h] * dinv + cb_ref[h] for h in (0, 1)]


def _accum_stats(p, i, ch, ssum, ssq):
    @pl.when(p == 0)
    def _():
        s1 = [jnp.sum(c, axis=0, keepdims=True) for c in ch]
        s2 = [jnp.sum(c * c, axis=0, keepdims=True) for c in ch]

        @pl.when(i == 0)
        def _():
            for h in (0, 1):
                ssum[h] = s1[h]
                ssq[h] = s2[h]

        @pl.when(i > 0)
        def _():
            for h in (0, 1):
                ssum[h] += s1[h]
                ssq[h] += s2[h]


def _normed_h(ch, ssum, ssq, nw_ref, nb_ref, nm_ref, res_ref):
    out = []
    for h in (0, 1):
        mean = ssum[h] * (1.0 / N)
        msq = ssq[h] * (1.0 / N)
        ctr = mean * nm_ref[h]
        var = msq - 2.0 * ctr * mean + ctr * ctr
        a = nw_ref[h] * lax.rsqrt(var + EPS)
        y = (ch[h] - ctr) * a + nb_ref[h]
        out.append(jnp.maximum(y, 0.0) + res_ref[h])
    return out


def _layer_body(agg_ref, dinv_ref, cb_ref, nw_ref, nb_ref, nm_ref, res_ref,
                wn_ref, h_ref, hs_ref, ssum, ssq):
    p = pl.program_id(0)
    i = pl.program_id(1)
    ch = _halves(agg_ref, dinv_ref, cb_ref)
    _accum_stats(p, i, ch, ssum, ssq)

    @pl.when(p == 1)
    def _():
        hn = _normed_h(ch, ssum, ssq, nw_ref, nb_ref, nm_ref, res_ref)
        h_ref[0] = hn[0]
        h_ref[1] = hn[1]
        w = wn_ref[...]
        hw = (jnp.dot(hn[0], w[:H], preferred_element_type=jnp.float32)
              + jnp.dot(hn[1], w[H:], preferred_element_type=jnp.float32))
        hs = hw * dinv_ref[...]
        hs_ref[0] = hs[:, :H]
        hs_ref[1] = hs[:, H:]


_VEC = pl.BlockSpec((2, 1, H), lambda p, i: (0, 0, 0))

_layer_call = pl.pallas_call(
    _layer_body,
    grid=(2, GRID),
    in_specs=[
        pl.BlockSpec((2, BN, H), lambda p, i: (0, i, 0)),
        pl.BlockSpec((BN, 1), lambda p, i: (i, 0)),
        _VEC, _VEC, _VEC, _VEC,
        pl.BlockSpec((2, BN, H), lambda p, i: (0, p * i, 0)),
        pl.BlockSpec((D, D), lambda p, i: (0, 0)),
    ],
    out_specs=[
        pl.BlockSpec((2, BN, H), lambda p, i: (0, p * i, 0)),
        pl.BlockSpec((2, BN, H), lambda p, i: (0, p * i, 0)),
    ],
    out_shape=[
        jax.ShapeDtypeStruct((2, N, H), jnp.float32),
        jax.ShapeDtypeStruct((2, N, H), jnp.float32),
    ],
    scratch_shapes=[
        pltpu.VMEM((2, 1, H), jnp.float32),
        pltpu.VMEM((2, 1, H), jnp.float32),
    ],
)


def _final_body(agg_ref, dinv_ref, cb_ref, nw_ref, nb_ref, nm_ref, res_ref,
                l1_ref, l1b_ref, l2_ref, l2b_ref, out_ref, ssum, ssq):
    p = pl.program_id(0)
    i = pl.program_id(1)
    ch = _halves(agg_ref, dinv_ref, cb_ref)
    _accum_stats(p, i, ch, ssum, ssq)

    @pl.when(p == 1)
    def _():
        hn = _normed_h(ch, ssum, ssq, nw_ref, nb_ref, nm_ref, res_ref)
        l1 = l1_ref[...]
        t = jnp.maximum(
            jnp.dot(hn[0], l1[:H], preferred_element_type=jnp.float32)
            + jnp.dot(hn[1], l1[H:], preferred_element_type=jnp.float32)
            + l1b_ref[...], 0.0)
        out_ref[...] = (
            jnp.dot(t, l2_ref[...], preferred_element_type=jnp.float32)
            + l2b_ref[...])


_final_call = pl.pallas_call(
    _final_body,
    grid=(2, GRID),
    in_specs=[
        pl.BlockSpec((2, BN, H), lambda p, i: (0, i, 0)),
        pl.BlockSpec((BN, 1), lambda p, i: (i, 0)),
        _VEC, _VEC, _VEC, _VEC,
        pl.BlockSpec((2, BN, H), lambda p, i: (0, p * i, 0)),
        pl.BlockSpec((D, D), lambda p, i: (0, 0)),
        pl.BlockSpec((1, D), lambda p, i: (0, 0)),
        pl.BlockSpec((D, OUT), lambda p, i: (0, 0)),
        pl.BlockSpec((1, OUT), lambda p, i: (0, 0)),
    ],
    out_specs=[pl.BlockSpec((BN, OUT), lambda p, i: (p * i, 0))],
    out_shape=[jax.ShapeDtypeStruct((N, OUT), jnp.float32)],
    scratch_shapes=[
        pltpu.VMEM((2, 1, H), jnp.float32),
        pltpu.VMEM((2, 1, H), jnp.float32),
    ],
)


def kernel(x, edge_index, conv_Ws, conv_bs, norm_ws, norm_bs, norm_ms,
           proj_W, lin_Ws, lin_bs):
    src = edge_index[0].astype(jnp.int32)
    dst = edge_index[1].astype(jnp.int32)
    ar = jnp.arange(N, dtype=jnp.int32)
    pad = E_PAD - E_TOT
    src_p = jnp.concatenate([src, ar, jnp.zeros((pad,), jnp.int32)])
    dst_p = jnp.concatenate([dst, ar, jnp.full((pad,), N, jnp.int32)])
    srcs = jnp.concatenate([src_p, src_p + N])

    _deg_call, _agg_call = _sc_calls()
    deg2 = _deg_call(dst_p).reshape(NC, NROWS, 16)
    hs, hproj, dinv = _prep_call(x, conv_Ws[0], proj_W, deg2)

    srcs2 = srcs.reshape(NC * NS, AGG_NB, K)
    dst2 = dst_p.reshape(NS, AGG_NB, K)

    res = hproj
    out = None
    for i in range(3):
        agg = _agg_call(hs.reshape(NC * N, H), srcs2, dst2)
        agg = agg.reshape(NC, NROWS, H)
        cb = conv_bs[i].reshape(2, 1, H)
        nw = norm_ws[i].reshape(2, 1, H)
        nb = norm_bs[i].reshape(2, 1, H)
        nm = norm_ms[i].reshape(2, 1, H)
        if i < 2:
            res, hs = _layer_call(agg, dinv, cb, nw, nb, nm, res,
                                  conv_Ws[i + 1])
        else:
            (out,) = _final_call(agg, dinv, cb, nw, nb, nm, res,
                                 lin_Ws[0], lin_bs[0].reshape(1, D),
                                 lin_Ws[1], lin_bs[1].reshape(1, OUT))
    return out
